# Initial kernel scaffold; baseline (speedup 1.0000x reference)
import jax
import jax.numpy as jnp
from jax import lax
from jax.experimental import pallas as pl
from jax.experimental.pallas import tpu as pltpu
from jax.experimental.pallas import tpu_sc as plsc

_B = 64
_N = 512
_F = 64
_E = 16384
_NCONV = 6
_BB = 8
_ROWS_PER_TILE = _N // 32
_KC = 2048


def _sc_scatter_body(ei_hbm, w_hbm, out_hbm, rows_v, cols_v, w_v, blk_v):
    wid = lax.axis_index("s") * 2 + lax.axis_index("c")
    base = wid * _ROWS_PER_TILE
    pltpu.sync_copy(ei_hbm.at[0], rows_v)
    pltpu.sync_copy(ei_hbm.at[1], cols_v)
    pltpu.sync_copy(w_hbm, w_v)

    zero16 = jnp.zeros((16,), jnp.float32)

    @pl.loop(0, _ROWS_PER_TILE * _N // 16)
    def _zero(i):
        blk_v[pl.ds(i * 16, 16)] = zero16

    @pl.loop(0, _E // 16)
    def _scan(i):
        r = rows_v[pl.ds(i * 16, 16)]
        c = cols_v[pl.ds(i * 16, 16)]
        w = w_v[pl.ds(i * 16, 16)]
        w = jnp.where(r == c, 0.0, w)
        m = (r >= base) & (r < base + _ROWS_PER_TILE)
        li = (r - base) * _N + c
        li = jnp.where(m, li, 0)
        plsc.addupdate_scatter(blk_v, [li], w, mask=m)

    pltpu.sync_copy(blk_v, out_hbm.at[pl.ds(base * _N, _ROWS_PER_TILE * _N)])


def _build_adjacency(edge_index, edge_weight):
    fn = pl.kernel(
        _sc_scatter_body,
        out_type=jax.ShapeDtypeStruct((_N * _N,), jnp.float32),
        mesh=plsc.VectorSubcoreMesh(
            core_axis_name="c", subcore_axis_name="s",
            num_cores=2, num_subcores=16,
        ),
        scratch_types=[
            pltpu.VMEM((_E,), jnp.int32),
            pltpu.VMEM((_E,), jnp.int32),
            pltpu.VMEM((_E,), jnp.float32),
            pltpu.VMEM((_ROWS_PER_TILE * _N,), jnp.float32),
        ],
    )
    return fn(edge_index, edge_weight).reshape(_N, _N)


def _conv_body(xT_ref, at_ref, wa_ref, wb_ref, bias_ref, out_ref):
    at = at_ref[...]
    deg = jnp.sum(at, axis=1)
    pos = deg > 0.0
    dis = jnp.where(pos, lax.rsqrt(jnp.where(pos, deg, 1.0)), 0.0)
    lt = (-dis[:, None] * dis[None, :]) * at

    g = xT_ref[...].reshape(_BB * _F, _N)
    for l in range(_NCONV):
        wa = wa_ref[l]
        wb = wb_ref[l]
        bias = bias_ref[l][:, None]
        dn = (((0,), (0,)), ((), ()))
        zs = []
        gs = []
        for b in range(_BB):
            gb = g[b * _F:(b + 1) * _F, :]
            zs.append(lax.dot_general(wb, gb, dn))
            gs.append(lax.dot_general(wa, gb, dn) + bias)
        z = jnp.concatenate(zs, axis=0)
        g = jnp.concatenate(gs, axis=0) + jnp.dot(z, lt)
        if l < _NCONV - 1:
            g = jnp.maximum(g, 0.0)
    out_ref[...] = g.reshape(_BB, _F, _N)


def _run_conv(xT, at, wa_s, wb_s, bias_s):
    return pl.pallas_call(
        _conv_body,
        out_shape=jax.ShapeDtypeStruct((_B, _F, _N), jnp.float32),
        grid=(_B // _BB,),
        in_specs=[
            pl.BlockSpec((_BB, _F, _N), lambda i: (i, 0, 0)),
            pl.BlockSpec((_N, _N), lambda i: (0, 0)),
            pl.BlockSpec((_NCONV, _F, _F), lambda i: (0, 0, 0)),
            pl.BlockSpec((_NCONV, _F, _F), lambda i: (0, 0, 0)),
            pl.BlockSpec((_NCONV, _F), lambda i: (0, 0)),
        ],
        out_specs=pl.BlockSpec((_BB, _F, _N), lambda i: (i, 0, 0)),
    )(xT, at, wa_s, wb_s, bias_s)


def _cls_body(g_ref, w1_ref, cb1_ref, w2_ref, cb2_ref, w3_ref, cb3_ref,
              out_ref, acc_ref):
    k = pl.program_id(0)

    @pl.when(k == 0)
    def _():
        acc_ref[...] = jnp.zeros_like(acc_ref)

    acc_ref[...] += jnp.dot(g_ref[...], w1_ref[...])

    @pl.when(k == pl.num_programs(0) - 1)
    def _():
        h1 = acc_ref[...] + cb1_ref[...]
        h2 = jnp.dot(h1, w2_ref[...]) + cb2_ref[...]
        h3 = jnp.dot(h2, w3_ref[...]) + cb3_ref[...]
        out_ref[...] = h3


def _run_classifier(gflat, w1p, cb1, cw2, cb2, cw3, cb3):
    d0 = _N * _F
    nk = d0 // _KC
    nc = cw3.shape[1]
    return pl.pallas_call(
        _cls_body,
        out_shape=jax.ShapeDtypeStruct((_B, nc), jnp.float32),
        grid=(nk,),
        in_specs=[
            pl.BlockSpec((_B, _KC), lambda k: (0, k)),
            pl.BlockSpec((_KC, 256), lambda k: (k, 0)),
            pl.BlockSpec((1, 256), lambda k: (0, 0)),
            pl.BlockSpec((256, 128), lambda k: (0, 0)),
            pl.BlockSpec((1, 128), lambda k: (0, 0)),
            pl.BlockSpec((128, nc), lambda k: (0, 0)),
            pl.BlockSpec((1, nc), lambda k: (0, 0)),
        ],
        out_specs=pl.BlockSpec((_B, nc), lambda k: (0, 0)),
        scratch_shapes=[pltpu.VMEM((_B, 256), jnp.float32)],
    )(gflat, w1p, cb1, cw2, cb2, cw3, cb3)


def kernel(x, edge_index, edge_weight,
           W0a, W0b, b0, W1a, W1b, b1, W2a, W2b, b2,
           W3a, W3b, b3, W4a, W4b, b4, W5a, W5b, b5,
           cW1, cb1, cW2, cb2, cW3, cb3):
    ei = edge_index.astype(jnp.int32)
    at = _build_adjacency(ei, edge_weight)

    xT = x.transpose(0, 2, 1)
    wa_s = jnp.stack([W0a, W1a, W2a, W3a, W4a, W5a])
    wb_s = jnp.stack([W0b, W1b, W2b, W3b, W4b, W5b])
    bias_s = jnp.stack([b0, b1, b2, b3, b4, b5])
    g6 = _run_conv(xT, at, wa_s, wb_s, bias_s)

    gflat = g6.reshape(_B, _F * _N)
    w1p = cW1.reshape(_N, _F, 256).transpose(1, 0, 2).reshape(_F * _N, 256)
    return _run_classifier(gflat, w1p,
                           cb1.reshape(1, -1), cW2, cb2.reshape(1, -1),
                           cW3, cb3.reshape(1, -1))


# speedup vs baseline: 193.3666x; 193.3666x over previous
import jax
import jax.numpy as jnp
from jax import lax
from jax.experimental import pallas as pl
from jax.experimental.pallas import tpu as pltpu
from jax.experimental.pallas import tpu_sc as plsc

_B = 64
_N = 512
_F = 64
_E = 16384
_NCONV = 6
_BB = 8
_ROWS_PER_TILE = _N // 32
_KC = 2048


def _sc_scatter_body(ei_hbm, w_hbm, out_hbm, rows_v, cols_v, w_v, blk_v):
    wid = lax.axis_index("s") * 2 + lax.axis_index("c")
    base = wid * _ROWS_PER_TILE
    pltpu.sync_copy(ei_hbm.at[0], rows_v)
    pltpu.sync_copy(ei_hbm.at[1], cols_v)
    pltpu.sync_copy(w_hbm, w_v)

    zero16 = jnp.zeros((16,), jnp.float32)

    @pl.loop(0, _ROWS_PER_TILE * _N // 16)
    def _zero(i):
        blk_v[pl.ds(i * 16, 16)] = zero16

    @pl.loop(0, _E // 16)
    def _scan(i):
        r = rows_v[pl.ds(i * 16, 16)]
        c = cols_v[pl.ds(i * 16, 16)]
        w = w_v[pl.ds(i * 16, 16)]
        w = jnp.where(r == c, 0.0, w)
        m = (r >= base) & (r < base + _ROWS_PER_TILE)
        li = (r - base) * _N + c
        li = jnp.where(m, li, 0)
        plsc.addupdate_scatter(blk_v, [li], w, mask=m)

    pltpu.sync_copy(blk_v, out_hbm.at[pl.ds(base * _N, _ROWS_PER_TILE * _N)])


def _build_adjacency(edge_index, edge_weight):
    fn = pl.kernel(
        _sc_scatter_body,
        out_type=jax.ShapeDtypeStruct((_N * _N,), jnp.float32),
        mesh=plsc.VectorSubcoreMesh(
            core_axis_name="c", subcore_axis_name="s",
            num_cores=2, num_subcores=16,
        ),
        scratch_types=[
            pltpu.VMEM((_E,), jnp.int32),
            pltpu.VMEM((_E,), jnp.int32),
            pltpu.VMEM((_E,), jnp.float32),
            pltpu.VMEM((_ROWS_PER_TILE * _N,), jnp.float32),
        ],
        compiler_params=pltpu.CompilerParams(needs_layout_passes=False),
    )
    return fn(edge_index, edge_weight).reshape(_N, _N)


def _conv_body(xT_ref, at_ref, wa_ref, wb_ref, bias_ref, out_ref):
    at = at_ref[...]
    deg = jnp.sum(at, axis=1)
    pos = deg > 0.0
    dis = jnp.where(pos, lax.rsqrt(jnp.where(pos, deg, 1.0)), 0.0)
    lt = (-dis[:, None] * dis[None, :]) * at

    g = xT_ref[...].reshape(_BB * _F, _N)
    for l in range(_NCONV):
        wa = wa_ref[l]
        wb = wb_ref[l]
        bias = bias_ref[l][:, None]
        dn = (((0,), (0,)), ((), ()))
        zs = []
        gs = []
        for b in range(_BB):
            gb = g[b * _F:(b + 1) * _F, :]
            zs.append(lax.dot_general(wb, gb, dn))
            gs.append(lax.dot_general(wa, gb, dn) + bias)
        z = jnp.concatenate(zs, axis=0)
        g = jnp.concatenate(gs, axis=0) + jnp.dot(z, lt)
        if l < _NCONV - 1:
            g = jnp.maximum(g, 0.0)
    out_ref[...] = g.reshape(_BB, _F, _N)


def _run_conv(xT, at, wa_s, wb_s, bias_s):
    return pl.pallas_call(
        _conv_body,
        out_shape=jax.ShapeDtypeStruct((_B, _F, _N), jnp.float32),
        grid=(_B // _BB,),
        in_specs=[
            pl.BlockSpec((_BB, _F, _N), lambda i: (i, 0, 0)),
            pl.BlockSpec((_N, _N), lambda i: (0, 0)),
            pl.BlockSpec((_NCONV, _F, _F), lambda i: (0, 0, 0)),
            pl.BlockSpec((_NCONV, _F, _F), lambda i: (0, 0, 0)),
            pl.BlockSpec((_NCONV, _F), lambda i: (0, 0)),
        ],
        out_specs=pl.BlockSpec((_BB, _F, _N), lambda i: (i, 0, 0)),
    )(xT, at, wa_s, wb_s, bias_s)


def _cls_body(g_ref, w1_ref, cb1_ref, w2_ref, cb2_ref, w3_ref, cb3_ref,
              out_ref, acc_ref):
    k = pl.program_id(0)

    @pl.when(k == 0)
    def _():
        acc_ref[...] = jnp.zeros_like(acc_ref)

    acc_ref[...] += jnp.dot(g_ref[...], w1_ref[...])

    @pl.when(k == pl.num_programs(0) - 1)
    def _():
        h1 = acc_ref[...] + cb1_ref[...]
        h2 = jnp.dot(h1, w2_ref[...]) + cb2_ref[...]
        h3 = jnp.dot(h2, w3_ref[...]) + cb3_ref[...]
        out_ref[...] = h3


def _run_classifier(gflat, w1p, cb1, cw2, cb2, cw3, cb3):
    d0 = _N * _F
    nk = d0 // _KC
    nc = cw3.shape[1]
    return pl.pallas_call(
        _cls_body,
        out_shape=jax.ShapeDtypeStruct((_B, nc), jnp.float32),
        grid=(nk,),
        in_specs=[
            pl.BlockSpec((_B, _KC), lambda k: (0, k)),
            pl.BlockSpec((_KC, 256), lambda k: (k, 0)),
            pl.BlockSpec((1, 256), lambda k: (0, 0)),
            pl.BlockSpec((256, 128), lambda k: (0, 0)),
            pl.BlockSpec((1, 128), lambda k: (0, 0)),
            pl.BlockSpec((128, nc), lambda k: (0, 0)),
            pl.BlockSpec((1, nc), lambda k: (0, 0)),
        ],
        out_specs=pl.BlockSpec((_B, nc), lambda k: (0, 0)),
        scratch_shapes=[pltpu.VMEM((_B, 256), jnp.float32)],
    )(gflat, w1p, cb1, cw2, cb2, cw3, cb3)


def kernel(x, edge_index, edge_weight,
           W0a, W0b, b0, W1a, W1b, b1, W2a, W2b, b2,
           W3a, W3b, b3, W4a, W4b, b4, W5a, W5b, b5,
           cW1, cb1, cW2, cb2, cW3, cb3):
    ei = edge_index.astype(jnp.int32)
    at = _build_adjacency(ei, edge_weight)

    xT = x.transpose(0, 2, 1)
    wa_s = jnp.stack([W0a, W1a, W2a, W3a, W4a, W5a])
    wb_s = jnp.stack([W0b, W1b, W2b, W3b, W4b, W5b])
    bias_s = jnp.stack([b0, b1, b2, b3, b4, b5])
    g6 = _run_conv(xT, at, wa_s, wb_s, bias_s)

    gflat = g6.reshape(_B, _F * _N)
    w1p = cW1.reshape(_N, _F, 256).transpose(1, 0, 2).reshape(_F * _N, 256)
    return _run_classifier(gflat, w1p,
                           cb1.reshape(1, -1), cW2, cb2.reshape(1, -1),
                           cW3, cb3.reshape(1, -1))
